# Initial kernel scaffold; baseline (speedup 1.0000x reference)
import jax
import jax.numpy as jnp
from jax.experimental import pallas as pl


def kernel(x, edge_index, W_enc, conv_w, bn_gamma, bn_beta, srelu_bias, W_head):
    raise NotImplementedError("write your pallas kernel here")


# speedup vs baseline: 6.2832x; 6.2832x over previous
import functools

import jax
import jax.numpy as jnp
from jax import lax
from jax.experimental import pallas as pl
from jax.experimental.pallas import tpu as pltpu
from jax.experimental.pallas import tpu_sc as plsc

_BETA = 0.1
_C_MIN = 0.5
_RW = _C_MIN - _BETA
_ALPHA = 1.0 - _RW - _BETA

_NC = 2
_NS = 16
_LANES = 16
_NW = _NC * _NS
_CHUNK = 128
_DUMP = 128


def _ceil_div(a, b):
    return -(-a // b)


@functools.lru_cache(maxsize=None)
def _make_sc_scatter(n_pad, cpw, d, shift):
    half = n_pad // 2
    acc_rows = half + _DUMP
    rpt_acc = acc_rows // _NS
    rpt_out = half // _NS
    mask = (1 << shift) - 1
    mesh = plsc.VectorSubcoreMesh(core_axis_name="c", subcore_axis_name="s")

    @functools.partial(
        pl.kernel,
        out_type=jax.ShapeDtypeStruct((_NC, n_pad, d), jnp.float32),
        mesh=mesh,
        scratch_types=[
            pltpu.VMEM((cpw, _CHUNK), jnp.int32),
            pltpu.VMEM((cpw, _CHUNK), jnp.int32),
            pltpu.VMEM((cpw, _CHUNK), jnp.int32),
            pltpu.VMEM((cpw, _CHUNK), jnp.int32),
            pltpu.VMEM((_CHUNK, d), jnp.float32),
            pltpu.VMEM((_CHUNK, d), jnp.float32),
            pltpu.VMEM_SHARED((acc_rows, d), jnp.float32),
            pltpu.SemaphoreType.DMA,
        ],
    )
    def sc_scatter(pk_hbm, g_hbm, out_hbm,
                   pk_v, src_v, dst0_v, dst1_v, rows_v, zbuf, acc, sem):
        cid = lax.axis_index("c")
        sid = lax.axis_index("s")

        def zrow(r, carry):
            for c0 in range(d // _LANES):
                zbuf[r, pl.ds(c0 * _LANES, _LANES)] = jnp.zeros(
                    (_LANES,), jnp.float32)
            return carry

        lax.fori_loop(0, _CHUNK, zrow, 0)
        pltpu.sync_copy(pk_hbm.at[cid, sid], pk_v)

        def urow(j, carry):
            for c0 in range(_CHUNK // _LANES):
                sl = pl.ds(c0 * _LANES, _LANES)
                v = pk_v[j, sl]
                dst = jnp.right_shift(v, shift)
                dump = half + jnp.bitwise_and(dst, _DUMP - 1)
                rel1 = dst - half
                src_v[j, sl] = jnp.bitwise_and(v, mask)
                dst0_v[j, sl] = jnp.where(dst < half, dst, dump)
                dst1_v[j, sl] = jnp.where(rel1 >= 0, rel1, dump)
            return carry

        lax.fori_loop(0, cpw, urow, 0)

        for phase, dstp_v in ((0, dst0_v), (1, dst1_v)):

            def zcp(i, carry):
                pltpu.sync_copy(
                    zbuf, acc.at[pl.ds(sid * rpt_acc + i * _CHUNK, _CHUNK)])
                return carry

            nfull = rpt_acc // _CHUNK
            lax.fori_loop(0, nfull, zcp, 0)
            rem = rpt_acc - nfull * _CHUNK
            if rem:
                pltpu.sync_copy(
                    zbuf.at[pl.ds(0, rem)],
                    acc.at[pl.ds(sid * rpt_acc + nfull * _CHUNK, rem)])
            plsc.subcore_barrier()

            def step(j, carry):
                pltpu.async_copy(g_hbm.at[src_v.at[j]], rows_v, sem).wait()
                pltpu.sync_copy(rows_v, acc.at[dstp_v.at[j]], add=True)
                return carry

            lax.fori_loop(0, cpw, step, 0)
            plsc.subcore_barrier()
            pltpu.sync_copy(
                acc.at[pl.ds(sid * rpt_out, rpt_out)],
                out_hbm.at[cid,
                           pl.ds(phase * half + sid * rpt_out, rpt_out)])
            plsc.subcore_barrier()

    return sc_scatter


def _enc_body(x_ref, w_ref, degp_ref, h_ref, g_ref, dinv_ref, *, n):
    t = jnp.dot(x_ref[...], w_ref[...], preferred_element_type=jnp.float32)
    t = jnp.maximum(t, 0.0)
    deg = degp_ref[0, :n, 0:1] + degp_ref[1, :n, 0:1] + 1.0
    dv = lax.rsqrt(deg)
    h_ref[...] = t
    g_ref[...] = t * dv
    dinv_ref[...] = dv


def _layer_body(pa_ref, h_ref, x0_ref, dinv_ref, w_ref,
                gam_ref, bet_ref, sb_ref, hn_ref, gn_ref, *, n):
    agg = pa_ref[0, :n, :] + pa_ref[1, :n, :]
    dv = dinv_ref[...]
    h = h_ref[...]
    hi = agg * dv + h * (dv * dv)
    sup = _ALPHA * hi + _RW * h + _BETA * x0_ref[...]
    t = jnp.dot(sup, w_ref[...], preferred_element_type=jnp.float32)
    mean = jnp.mean(t, axis=0, keepdims=True)
    cm = t - mean
    var = jnp.mean(cm * cm, axis=0, keepdims=True)
    hbn = cm * lax.rsqrt(var + 1e-5) * gam_ref[...] + bet_ref[...]
    sb = sb_ref[...]
    hn = jnp.maximum(hbn - sb, 0.0) + sb
    hn_ref[...] = hn
    gn_ref[...] = hn * dv


def _head_body(h_ref, wh_ref, out_ref):
    out_ref[...] = jnp.dot(h_ref[...], wh_ref[...],
                           preferred_element_type=jnp.float32)


def kernel(x, edge_index, W_enc, conv_w, bn_gamma, bn_beta, srelu_bias,
           W_head):
    n, d = x.shape
    e = edge_index.shape[1]
    num_layers = conv_w.shape[0]
    cpw = _ceil_div(e, _NW * _CHUNK)
    e_pad = _NW * cpw * _CHUNK
    n_pad = 2 * _NS * _CHUNK * _ceil_div(n + 1, 2 * _NS * _CHUNK)
    shift = max(int(n_pad - 1).bit_length(), 1)
    assert 2 * shift <= 31, "packed src/dst index exceeds int32"

    src = edge_index[0].astype(jnp.int32)
    dst = edge_index[1].astype(jnp.int32)
    padn = e_pad - e
    padi = jnp.arange(padn, dtype=jnp.int32)
    src_p = jnp.concatenate([src, padi % n])
    dst_p = jnp.concatenate([dst, n + padi % (n_pad - n)])
    packed = (src_p | (dst_p << shift)).reshape(_NC, _NS, cpw, _CHUNK)

    sc_scatter = _make_sc_scatter(n_pad, cpw, d, shift)

    deg_parts = sc_scatter(packed, jnp.ones((n, d), jnp.float32))

    enc = pl.pallas_call(
        functools.partial(_enc_body, n=n),
        out_shape=[
            jax.ShapeDtypeStruct((n, d), jnp.float32),
            jax.ShapeDtypeStruct((n, d), jnp.float32),
            jax.ShapeDtypeStruct((n, 1), jnp.float32),
        ],
    )
    h0, g, dinv = enc(x, W_enc, deg_parts)

    layer = pl.pallas_call(
        functools.partial(_layer_body, n=n),
        out_shape=[
            jax.ShapeDtypeStruct((n, d), jnp.float32),
            jax.ShapeDtypeStruct((n, d), jnp.float32),
        ],
    )

    def body(carry, win):
        hc, gc = carry
        wl, gam, bet, sb = win
        parts = sc_scatter(packed, gc)
        hc, gc = layer(parts, hc, h0, dinv, wl, gam, bet, sb)
        return (hc, gc), None

    (h, _), _ = lax.scan(
        body, (h0, g),
        (conv_w, bn_gamma.reshape(num_layers, 1, d),
         bn_beta.reshape(num_layers, 1, d),
         srelu_bias.reshape(num_layers, 1, d)))

    head = pl.pallas_call(
        _head_body,
        out_shape=jax.ShapeDtypeStruct((n, d), jnp.float32),
    )
    return head(h, W_head)
